# baseline (speedup 1.0000x reference)
import functools

import jax
import jax.numpy as jnp
from jax import lax
from jax.experimental import pallas as pl
from jax.experimental.pallas import tpu as pltpu
from jax.experimental.pallas import tpu_sc as plsc

f32 = jnp.float32
i32 = jnp.int32

_N = 10000
_E = 320000
_NH = 128
_L = 3

_NC = 2
_NSUB = 16
_W = _NC * _NSUB
_CH = 128
_K = 79
_EP = _K * _CH
_EPAD = _W * _EP
_BE = 1024
_GE = _EPAD // _BE
_BN = 1000
_GN = _N // _BN
_NACC = 10112

def _sc_mesh():
    return plsc.VectorSubcoreMesh(core_axis_name="c", subcore_axis_name="s")


_K2 = _EPAD // (_CH * _NSUB)
_EP2 = _EPAD // _NSUB
_CS = 64
_KG = _EP2 // _CS


def _sc_gather(hrp, hcp, ridx_g, cidx_g):
    @functools.partial(
        pl.kernel,
        out_type=(
            jax.ShapeDtypeStruct((_EPAD, _NH), f32),
            jax.ShapeDtypeStruct((_EPAD, _NH), f32),
        ),
        mesh=_sc_mesh(),
        scratch_types=[
            pltpu.VMEM((_KG // 2, _CS), i32),
            pltpu.VMEM((_CS, _NH), f32),
            pltpu.VMEM((_CS, _NH), f32),
            pltpu.VMEM_SHARED((_N, _NH), f32),
            pltpu.SemaphoreType.DMA,
            pltpu.SemaphoreType.DMA,
            pltpu.SemaphoreType.DMA,
            pltpu.SemaphoreType.DMA,
        ],
    )
    def k(hrp_h, hcp_h, ridx_h, cidx_h, hr_o, hc_o,
          idx_v, hb0, hb1, spm_tab, sg0, sg1, sw0, sw1):
        c = lax.axis_index("c")
        s = lax.axis_index("s")

        @pl.when(jnp.logical_and(c == 0, s == 0))
        def _():
            pltpu.sync_copy(hrp_h, spm_tab)

        @pl.when(jnp.logical_and(c == 1, s == 0))
        def _():
            pltpu.sync_copy(hcp_h, spm_tab)

        plsc.subcore_barrier()
        nhalf = _KG // 2
        npair = nhalf // 2

        def emit(idx_src, out_h):
            for half in range(2):
                pltpu.sync_copy(idx_src.at[s, half], idx_v)
                base = s * _EP2 + half * nhalf * _CS
                pltpu.async_copy(spm_tab.at[idx_v.at[0]], hb0, sg0)

                def pair(jj, carry):
                    j0 = jj * 2
                    off0 = base + j0 * _CS
                    off1 = off0 + _CS
                    pltpu.make_async_copy(
                        spm_tab.at[idx_v.at[j0]], hb0, sg0).wait()

                    @pl.when(jj > 0)
                    def _():
                        pltpu.make_async_copy(
                            hb1, out_h.at[pl.ds(base, _CS)], sw1).wait()

                    pltpu.async_copy(spm_tab.at[idx_v.at[j0 + 1]], hb1, sg1)
                    pltpu.async_copy(hb0, out_h.at[pl.ds(off0, _CS)], sw0)
                    pltpu.make_async_copy(
                        spm_tab.at[idx_v.at[j0 + 1]], hb1, sg1).wait()
                    pltpu.make_async_copy(
                        hb0, out_h.at[pl.ds(base, _CS)], sw0).wait()

                    @pl.when(jj < npair - 1)
                    def _():
                        pltpu.async_copy(spm_tab.at[idx_v.at[j0 + 2]], hb0, sg0)

                    pltpu.async_copy(hb1, out_h.at[pl.ds(off1, _CS)], sw1)
                    return carry

                lax.fori_loop(0, npair, pair, 0)
                pltpu.make_async_copy(
                    hb1, out_h.at[pl.ds(base, _CS)], sw1).wait()

        @pl.when(c == 0)
        def _():
            emit(ridx_h, hr_o)

        @pl.when(c == 1)
        def _():
            emit(cidx_h, hc_o)

    return k(hrp, hcp, ridx_g, cidx_g)


def _sc_xgather(xflat, ridx16, cidx16):
    @functools.partial(
        pl.kernel,
        out_type=(
            jax.ShapeDtypeStruct((_EPAD * 4,), f32),
            jax.ShapeDtypeStruct((_EPAD * 4,), f32),
        ),
        mesh=_sc_mesh(),
        scratch_types=[
            pltpu.VMEM((_K2, _CH), i32),
            pltpu.VMEM((_N * 4,), f32),
            pltpu.VMEM((_CH * 8,), f32),
            pltpu.VMEM((_CH * 8,), f32),
            pltpu.SemaphoreType.DMA,
            pltpu.SemaphoreType.DMA,
        ],
        compiler_params=pltpu.CompilerParams(needs_layout_passes=False),
    )
    def k(x_h, ridx_h, cidx_h, xr_o, xc_o, idx_v, xtab, xb0, xb1, sx0, sx1):
        c = lax.axis_index("c")
        s = lax.axis_index("s")

        @pl.when(c == 0)
        def _():
            pltpu.sync_copy(ridx_h.at[s], idx_v)

        @pl.when(c == 1)
        def _():
            pltpu.sync_copy(cidx_h.at[s], idx_v)

        pltpu.sync_copy(x_h, xtab)
        base = s * _EP2
        lanes = lax.iota(i32, 16)
        cx = 2 * _CH
        nck = _EP2 // cx

        def xgather(j, xb):
            for half in range(2):
                row = idx_v.at[2 * j + half]
                for g in range(_CH // 16):
                    iv = row[pl.ds(g * 16, 16)] * 4
                    pos = (lanes + half * _CH + g * 16) * 4
                    for kk in range(4):
                        plsc.store_scatter(xb, [pos + kk],
                                           plsc.load_gather(xtab, [iv + kk]))

        def emit(out_x):
            def pair(jj, carry):
                j0 = jj * 2
                for j, xb, sx in ((j0, xb0, sx0), (j0 + 1, xb1, sx1)):
                    @pl.when(jj > 0)
                    def _():
                        pltpu.make_async_copy(
                            xb, out_x.at[pl.ds(base * 4, cx * 4)], sx).wait()

                    xgather(j, xb)
                    pltpu.async_copy(
                        xb, out_x.at[pl.ds((base + j * cx) * 4, cx * 4)], sx)
                return carry

            lax.fori_loop(0, nck // 2, pair, 0)
            pltpu.make_async_copy(
                xb0, out_x.at[pl.ds(base * 4, cx * 4)], sx0).wait()
            xgather(nck - 1, xb0)
            pltpu.async_copy(
                xb0, out_x.at[pl.ds((base + (nck - 1) * cx) * 4, cx * 4)], sx0)
            for xb, sx in ((xb0, sx0), (xb1, sx1)):
                pltpu.make_async_copy(
                    xb, out_x.at[pl.ds(base * 4, cx * 4)], sx).wait()

        @pl.when(c == 0)
        def _():
            emit(xr_o)

        @pl.when(c == 1)
        def _():
            emit(xc_o)

    return k(xflat, ridx16, cidx16)


def _scatter_loop(src_h, base, nchunks, sidx_v, acc, mb0, mb1, sl0, sl1):
    pltpu.async_copy(src_h.at[pl.ds(base, _CS)], mb0, sl0)
    npair = nchunks // 2

    def pair(jj, carry):
        j0 = jj * 2
        off0 = base + j0 * _CS
        pltpu.make_async_copy(src_h.at[pl.ds(off0, _CS)], mb0, sl0).wait()
        pltpu.async_copy(src_h.at[pl.ds(off0 + _CS, _CS)], mb1, sl1)
        pltpu.sync_copy(mb0, acc.at[sidx_v.at[j0]], add=True)
        pltpu.make_async_copy(src_h.at[pl.ds(off0, _CS)], mb1, sl1).wait()

        @pl.when(jj < npair - 1)
        def _():
            pltpu.async_copy(src_h.at[pl.ds(off0 + 2 * _CS, _CS)], mb0, sl0)

        pltpu.sync_copy(mb1, acc.at[sidx_v.at[j0 + 1]], add=True)
        return carry

    lax.fori_loop(0, npair, pair, 0)


def _sc_scatter_mt(m, trans_pad, sidx16, z128):
    @functools.partial(
        pl.kernel,
        out_type=(
            jax.ShapeDtypeStruct((_NACC, _NH), f32),
            jax.ShapeDtypeStruct((_NACC, _NH), f32),
        ),
        mesh=_sc_mesh(),
        scratch_types=[
            pltpu.VMEM((_KG // 2, _CS), i32),
            pltpu.VMEM((_CS, _NH), f32),
            pltpu.VMEM((_CS, _NH), f32),
            pltpu.VMEM_SHARED((_NACC, _NH), f32),
            pltpu.SemaphoreType.DMA,
            pltpu.SemaphoreType.DMA,
        ],
    )
    def k(m_h, t_h, sidx_h, z_h, p_o, q_o, sidx_v, mb0, mb1, acc, sl0, sl1):
        c = lax.axis_index("c")
        s = lax.axis_index("s")

        @pl.when(s == 0)
        def _():
            pltpu.sync_copy(z_h, acc)

        plsc.subcore_barrier()

        def halves(src_h):
            for half in range(2):
                pltpu.sync_copy(sidx_h.at[s, half], sidx_v)
                base = s * _EP2 + half * (_KG // 2) * _CS
                _scatter_loop(src_h, base, _KG // 2, sidx_v, acc,
                              mb0, mb1, sl0, sl1)

        @pl.when(c == 0)
        def _():
            halves(m_h)

        @pl.when(c == 1)
        def _():
            halves(t_h)

        plsc.subcore_barrier()

        seg = _NACC // _NSUB
        start = s * seg

        @pl.when(c == 0)
        def _():
            pltpu.sync_copy(acc.at[pl.ds(start, seg)], p_o.at[pl.ds(start, seg)])

        @pl.when(c == 1)
        def _():
            pltpu.sync_copy(acc.at[pl.ds(start, seg)], q_o.at[pl.ds(start, seg)])

    return k(m, trans_pad, sidx16, z128)


def _sc_scatter_m(m, sidx, z128):
    @functools.partial(
        pl.kernel,
        out_type=(
            jax.ShapeDtypeStruct((_NACC, _NH), f32),
            jax.ShapeDtypeStruct((_NACC, _NH), f32),
        ),
        mesh=_sc_mesh(),
        scratch_types=[
            pltpu.VMEM((_EP // _CS, _CS), i32),
            pltpu.VMEM((_CS, _NH), f32),
            pltpu.VMEM((_CS, _NH), f32),
            pltpu.VMEM_SHARED((_NACC, _NH), f32),
            pltpu.SemaphoreType.DMA,
            pltpu.SemaphoreType.DMA,
        ],
    )
    def k(m_h, sidx_h, z_h, p0_o, p1_o, sidx_v, mb0, mb1, acc, sl0, sl1):
        c = lax.axis_index("c")
        s = lax.axis_index("s")
        wid = s * _NC + c

        @pl.when(s == 0)
        def _():
            pltpu.sync_copy(z_h, acc)

        pltpu.sync_copy(sidx_h.at[wid], sidx_v)
        plsc.subcore_barrier()
        base = wid * _EP
        _scatter_loop(m_h, base, _EP // _CS, sidx_v, acc, mb0, mb1, sl0, sl1)
        plsc.subcore_barrier()

        seg = _NACC // _NSUB
        start = s * seg

        @pl.when(c == 0)
        def _():
            pltpu.sync_copy(acc.at[pl.ds(start, seg)], p0_o.at[pl.ds(start, seg)])

        @pl.when(c == 1)
        def _():
            pltpu.sync_copy(acc.at[pl.ds(start, seg)], p1_o.at[pl.ds(start, seg)])

    return k(m, sidx, z128)


def _psi(t):
    return jnp.sign(t) * jnp.log(jnp.abs(t) + 1.0)


def _dot3(a, w):
    bf16 = jnp.bfloat16
    ah = a.astype(bf16)
    al = (a - ah.astype(f32)).astype(bf16)
    wh = w.astype(bf16)
    wl = (w - wh.astype(f32)).astype(bf16)
    return (jnp.dot(ah, wh, preferred_element_type=f32)
            + jnp.dot(al, wh, preferred_element_type=f32)
            + jnp.dot(ah, wl, preferred_element_type=f32))


def _edge_a(hr_g, hc_g, xr, xc, ea, w4):
    def body(hr, hc, xr_r, xc_r, ea_r, w4_r, m1_o, st_o):
        b = pl.program_id(0)
        xrb = xr_r[...]
        xcb = xc_r[...]
        xd = xrb - xcb
        normsq = xd[:, 0:1] ** 2 - (xd[:, 1:2] ** 2 + xd[:, 2:3] ** 2 + xd[:, 3:4] ** 2)
        dotsq = xrb[:, 0:1] * xcb[:, 0:1] - (
            xrb[:, 1:2] * xcb[:, 1:2] + xrb[:, 2:3] * xcb[:, 2:3] + xrb[:, 3:4] * xcb[:, 3:4])
        norms = _psi(normsq)
        dots = _psi(dotsq)
        eab = ea_r[...]
        w = w4_r[...]
        eterm = (eab[:, 0:1] * w[0:1, :] + eab[:, 1:2] * w[1:2, :]
                 + norms * w[2:3, :] + dots * w[3:4, :])
        m1 = hr[...] + hc[...] + eterm
        m1_o[...] = m1
        valid = (lax.broadcasted_iota(i32, (_BE, 1), 0) + b * _BE) < _E
        m1m = jnp.where(valid, m1, 0.0)
        ssum = jnp.sum(m1m, axis=0, keepdims=True)
        ssq = jnp.sum(m1m * m1m, axis=0, keepdims=True)
        upd = jnp.concatenate([ssum, ssq, jnp.zeros((6, _NH), f32)], axis=0)

        @pl.when(b == 0)
        def _():
            st_o[...] = jnp.zeros((8, _NH), f32)

        st_o[...] += upd

    return pl.pallas_call(
        body,
        grid=(_GE,),
        in_specs=[
            pl.BlockSpec((_BE, _NH), lambda b: (b, 0)),
            pl.BlockSpec((_BE, _NH), lambda b: (b, 0)),
            pl.BlockSpec((_BE, 4), lambda b: (b, 0)),
            pl.BlockSpec((_BE, 4), lambda b: (b, 0)),
            pl.BlockSpec((_BE, 2), lambda b: (b, 0)),
            pl.BlockSpec((4, _NH), lambda b: (0, 0)),
        ],
        out_specs=[
            pl.BlockSpec((_BE, _NH), lambda b: (b, 0)),
            pl.BlockSpec((8, _NH), lambda b: (0, 0)),
        ],
        out_shape=[
            jax.ShapeDtypeStruct((_EPAD, _NH), f32),
            jax.ShapeDtypeStruct((8, _NH), f32),
        ],
    )(hr_g, hc_g, xr, xc, ea, w4)


def _edge_b(m1, stats, gamma, beta, We2, be2, Wm, bm, xr, xc, Wx1, bx1, Wx2, last):
    def body(*refs):
        if last:
            (m1_r, st_r, g_r, b_r, We2_r, be2_r, Wm_r, bm_r, m_o) = refs
        else:
            (m1_r, st_r, g_r, b_r, We2_r, be2_r, Wm_r, bm_r,
             xr_r, xc_r, Wx1_r, bx1_r, Wx2_r, m_o, tr_o) = refs
        inv = 1.0 / float(_E)
        ssum = st_r[0:1, :]
        ssq = st_r[1:2, :]
        mu = ssum * inv
        var = ssq * inv - mu * mu
        scale = g_r[...] * lax.rsqrt(var + 1e-5)
        shift = b_r[...] - mu * scale
        e1 = jnp.maximum(m1_r[...] * scale + shift, 0.0)
        m2 = jnp.maximum(_dot3(e1, We2_r[...]) + be2_r[...], 0.0)
        wgt = jax.nn.sigmoid(
            jnp.dot(m2, Wm_r[...], preferred_element_type=f32) + bm_r[...])
        mv = m2 * wgt
        m_o[...] = mv
        if not last:
            t1 = jnp.maximum(_dot3(mv, Wx1_r[...]) + bx1_r[...], 0.0)
            t = jnp.dot(t1, Wx2_r[...], preferred_element_type=f32)
            tr = (xr_r[...] - xc_r[...]) * t
            tr_o[...] = jnp.concatenate(
                [tr, jnp.zeros((_BE, _NH - 4), f32)], axis=1)

    in_specs = [
        pl.BlockSpec((_BE, _NH), lambda b: (b, 0)),
        pl.BlockSpec((8, _NH), lambda b: (0, 0)),
        pl.BlockSpec((1, _NH), lambda b: (0, 0)),
        pl.BlockSpec((1, _NH), lambda b: (0, 0)),
        pl.BlockSpec((_NH, _NH), lambda b: (0, 0)),
        pl.BlockSpec((1, _NH), lambda b: (0, 0)),
        pl.BlockSpec((_NH, 1), lambda b: (0, 0)),
        pl.BlockSpec((1, 1), lambda b: (0, 0)),
    ]
    out_specs = [pl.BlockSpec((_BE, _NH), lambda b: (b, 0))]
    out_shape = [jax.ShapeDtypeStruct((_EPAD, _NH), f32)]
    args = [m1, stats, gamma, beta, We2, be2, Wm, bm]
    if not last:
        in_specs += [
            pl.BlockSpec((_BE, 4), lambda b: (b, 0)),
            pl.BlockSpec((_BE, 4), lambda b: (b, 0)),
            pl.BlockSpec((_NH, _NH), lambda b: (0, 0)),
            pl.BlockSpec((1, _NH), lambda b: (0, 0)),
            pl.BlockSpec((_NH, 1), lambda b: (0, 0)),
        ]
        out_specs += [pl.BlockSpec((_BE, _NH), lambda b: (b, 0))]
        out_shape += [jax.ShapeDtypeStruct((_EPAD, _NH), f32)]
        args += [xr, xc, Wx1, bx1, Wx2]
    res = pl.pallas_call(
        body, grid=(_GE,), in_specs=in_specs, out_specs=out_specs,
        out_shape=out_shape)(*args)
    return res[0] if last else (res[0], res[1])


def _embed(scalars, Wemb, bemb, We1r0, We1c0):
    def body(s_r, We_r, be_r, Wr_r, Wc_r, h_o, hr_o, hc_o):
        h = jnp.dot(s_r[...], We_r[...], preferred_element_type=f32) + be_r[...]
        h_o[...] = h
        hr_o[...] = _dot3(h, Wr_r[...])
        hc_o[...] = _dot3(h, Wc_r[...])

    return pl.pallas_call(
        body,
        grid=(_GN,),
        in_specs=[
            pl.BlockSpec((_BN, 16), lambda b: (b, 0)),
            pl.BlockSpec((16, _NH), lambda b: (0, 0)),
            pl.BlockSpec((1, _NH), lambda b: (0, 0)),
            pl.BlockSpec((_NH, _NH), lambda b: (0, 0)),
            pl.BlockSpec((_NH, _NH), lambda b: (0, 0)),
        ],
        out_specs=[
            pl.BlockSpec((_BN, _NH), lambda b: (b, 0)),
            pl.BlockSpec((_BN, _NH), lambda b: (b, 0)),
            pl.BlockSpec((_BN, _NH), lambda b: (b, 0)),
        ],
        out_shape=[
            jax.ShapeDtypeStruct((_N, _NH), f32),
            jax.ShapeDtypeStruct((_N, _NH), f32),
            jax.ShapeDtypeStruct((_N, _NH), f32),
        ],
    )(scalars, Wemb, bemb, We1r0, We1c0)


def _node1(h, parts, Wh1a, Wh1b, bh1):
    np_ = len(parts)

    def body(*refs):
        h_r = refs[0]
        p_rs = refs[1:1 + np_]
        Wa_r, Wb_r, b_r, z_o, st_o = refs[1 + np_:]
        b = pl.program_id(0)
        hagg = p_rs[0][...]
        for pr in p_rs[1:]:
            hagg = hagg + pr[...]
        z = _dot3(h_r[...], Wa_r[...]) + _dot3(hagg, Wb_r[...]) + b_r[...]
        z_o[...] = z
        ssum = jnp.sum(z, axis=0, keepdims=True)
        ssq = jnp.sum(z * z, axis=0, keepdims=True)
        upd = jnp.concatenate([ssum, ssq, jnp.zeros((6, _NH), f32)], axis=0)

        @pl.when(b == 0)
        def _():
            st_o[...] = jnp.zeros((8, _NH), f32)

        st_o[...] += upd

    return pl.pallas_call(
        body,
        grid=(_GN,),
        in_specs=(
            [pl.BlockSpec((_BN, _NH), lambda b: (b, 0))] * (1 + np_)
            + [
                pl.BlockSpec((_NH, _NH), lambda b: (0, 0)),
                pl.BlockSpec((_NH, _NH), lambda b: (0, 0)),
                pl.BlockSpec((1, _NH), lambda b: (0, 0)),
            ]
        ),
        out_specs=[
            pl.BlockSpec((_BN, _NH), lambda b: (b, 0)),
            pl.BlockSpec((8, _NH), lambda b: (0, 0)),
        ],
        out_shape=[
            jax.ShapeDtypeStruct((_N, _NH), f32),
            jax.ShapeDtypeStruct((8, _NH), f32),
        ],
    )(h, *parts, Wh1a, Wh1b, bh1)


def _node2(z, stats, gh, bh, Wh2, bh2, h, x, xq, We1r_n, We1c_n, last):
    def body(*refs):
        if last:
            (z_r, st_r, g_r, b_r, W2_r, b2_r, h_r, h_o) = refs
        else:
            (z_r, st_r, g_r, b_r, W2_r, b2_r, h_r, x_r, xq_r,
             Wr_r, Wc_r, h_o, hr_o, hc_o, x_o) = refs
        inv = 1.0 / float(_N)
        mu = st_r[0:1, :] * inv
        var = st_r[1:2, :] * inv - mu * mu
        scale = g_r[...] * lax.rsqrt(var + 1e-5)
        shift = b_r[...] - mu * scale
        hh = jnp.maximum(z_r[...] * scale + shift, 0.0)
        hn = h_r[...] + _dot3(hh, W2_r[...]) + b2_r[...]
        h_o[...] = hn
        if not last:
            hr_o[...] = _dot3(hn, Wr_r[...])
            hc_o[...] = _dot3(hn, Wc_r[...])
            x_o[...] = x_r[...] + xq_r[...]

    in_specs = [
        pl.BlockSpec((_BN, _NH), lambda b: (b, 0)),
        pl.BlockSpec((8, _NH), lambda b: (0, 0)),
        pl.BlockSpec((1, _NH), lambda b: (0, 0)),
        pl.BlockSpec((1, _NH), lambda b: (0, 0)),
        pl.BlockSpec((_NH, _NH), lambda b: (0, 0)),
        pl.BlockSpec((1, _NH), lambda b: (0, 0)),
        pl.BlockSpec((_BN, _NH), lambda b: (b, 0)),
    ]
    out_specs = [pl.BlockSpec((_BN, _NH), lambda b: (b, 0))]
    out_shape = [jax.ShapeDtypeStruct((_N, _NH), f32)]
    args = [z, stats, gh, bh, Wh2, bh2, h]
    if not last:
        in_specs += [
            pl.BlockSpec((_BN, 4), lambda b: (b, 0)),
            pl.BlockSpec((_BN, 4), lambda b: (b, 0)),
            pl.BlockSpec((_NH, _NH), lambda b: (0, 0)),
            pl.BlockSpec((_NH, _NH), lambda b: (0, 0)),
        ]
        out_specs += [
            pl.BlockSpec((_BN, _NH), lambda b: (b, 0)),
            pl.BlockSpec((_BN, _NH), lambda b: (b, 0)),
            pl.BlockSpec((_BN, 4), lambda b: (b, 0)),
        ]
        out_shape += [
            jax.ShapeDtypeStruct((_N, _NH), f32),
            jax.ShapeDtypeStruct((_N, _NH), f32),
            jax.ShapeDtypeStruct((_N, 4), f32),
        ]
        args += [x, xq, We1r_n, We1c_n]
    res = pl.pallas_call(
        body, grid=(_GN,), in_specs=in_specs, out_specs=out_specs,
        out_shape=out_shape)(*args)
    return res[0] if last else (res[0], res[1], res[2], res[3])


def _head(x, h, wxo, who, bo):
    def body(x_r, h_r, wx_r, wh_r, b_r, o_o):
        o = (jnp.dot(x_r[...], wx_r[...], preferred_element_type=f32)
             + jnp.dot(h_r[...], wh_r[...], preferred_element_type=f32) + b_r[...])
        o_o[...] = jax.nn.sigmoid(o)

    return pl.pallas_call(
        body,
        grid=(_GN,),
        in_specs=[
            pl.BlockSpec((_BN, 4), lambda b: (b, 0)),
            pl.BlockSpec((_BN, _NH), lambda b: (b, 0)),
            pl.BlockSpec((4, 1), lambda b: (0, 0)),
            pl.BlockSpec((_NH, 1), lambda b: (0, 0)),
            pl.BlockSpec((1, 1), lambda b: (0, 0)),
        ],
        out_specs=pl.BlockSpec((_BN, 1), lambda b: (b, 0)),
        out_shape=jax.ShapeDtypeStruct((_N, 1), f32),
    )(x, h, wxo, who, bo)


def _r2(v):
    return v.reshape(1, -1)


def kernel(scalars, x, edge_index, edge_attr, params):
    row = edge_index[0].astype(i32)
    col = edge_index[1].astype(i32)
    padz = jnp.zeros((_EPAD - _E,), i32)
    rpad = jnp.concatenate([row, padz])
    cpad = jnp.concatenate([col, padz])
    ridx16 = rpad.reshape(_NSUB, _K2, _CH)
    cidx16 = cpad.reshape(_NSUB, _K2, _CH)
    ridx_g = rpad.reshape(_NSUB, 2, _KG // 2, _CS)
    cidx_g = cpad.reshape(_NSUB, 2, _KG // 2, _CS)
    spad = jnp.concatenate([row, jnp.full((_EPAD - _E,), _N, i32)])
    sidx = spad.reshape(_W, _EP // _CS, _CS)
    sidx16 = spad.reshape(_NSUB, 2, _KG // 2, _CS)
    ea_pad = jnp.concatenate(
        [edge_attr, jnp.zeros((_EPAD - _E, 2), f32)], axis=0)
    z128 = jnp.zeros((_NACC, _NH), f32)

    We1 = params['We1']
    Wh1 = params['Wh1']

    h, hrp, hcp = _embed(scalars, params['Wemb'], _r2(params['bemb']),
                         We1[0, :_NH], We1[0, _NH:2 * _NH])

    for i in range(_L):
        last = i == _L - 1
        hr_g, hc_g = _sc_gather(hrp, hcp, ridx_g, cidx_g)
        xrf, xcf = _sc_xgather(x.reshape(-1), ridx16, cidx16)
        xr = xrf.reshape(_EPAD, 4)
        xc = xcf.reshape(_EPAD, 4)
        m1, st_e = _edge_a(hr_g, hc_g, xr, xc, ea_pad, We1[i, 2 * _NH:])
        if last:
            m = _edge_b(m1, st_e, _r2(params['bn_e_g'][i]), _r2(params['bn_e_b'][i]),
                        params['We2'][i], _r2(params['be2'][i]),
                        params['Wm'][i], _r2(params['bm'][i]),
                        None, None, None, None, None, True)
            p0, p1 = _sc_scatter_m(m, sidx, z128)
            parts = [p0[:_N], p1[:_N]]
        else:
            m, trans = _edge_b(m1, st_e, _r2(params['bn_e_g'][i]), _r2(params['bn_e_b'][i]),
                               params['We2'][i], _r2(params['be2'][i]),
                               params['Wm'][i], _r2(params['bm'][i]),
                               xr, xc, params['Wx1'][i], _r2(params['bx1'][i]),
                               params['Wx2'][i], False)
            p, q = _sc_scatter_mt(m, trans, sidx16, z128)
            parts = [p[:_N]]
        z, st_n = _node1(h, parts, Wh1[i, :_NH], Wh1[i, _NH:],
                         _r2(params['bh1'][i]))
        if last:
            h = _node2(z, st_n, _r2(params['bn_h_g'][i]), _r2(params['bn_h_b'][i]),
                       params['Wh2'][i], _r2(params['bh2'][i]), h,
                       None, None, None, None, True)
        else:
            h, hrp, hcp, x = _node2(
                z, st_n, _r2(params['bn_h_g'][i]), _r2(params['bn_h_b'][i]),
                params['Wh2'][i], _r2(params['bh2'][i]), h,
                x, q[:_N, 0:4], We1[i + 1, :_NH], We1[i + 1, _NH:2 * _NH],
                False)

    return _head(x, h, params['Wout'][:4], params['Wout'][4:],
                 _r2(params['bout']))


# speedup vs baseline: 1.0348x; 1.0348x over previous
import functools

import jax
import jax.numpy as jnp
from jax import lax
from jax.experimental import pallas as pl
from jax.experimental.pallas import tpu as pltpu
from jax.experimental.pallas import tpu_sc as plsc

f32 = jnp.float32
i32 = jnp.int32

_N = 10000
_E = 320000
_NH = 128
_L = 3

_NC = 2
_NSUB = 16
_W = _NC * _NSUB
_CH = 128
_K = 79
_EP = _K * _CH
_EPAD = _W * _EP
_BE = 1024
_GE = _EPAD // _BE
_BN = 1000
_GN = _N // _BN
_NACC = 10112

def _sc_mesh():
    return plsc.VectorSubcoreMesh(core_axis_name="c", subcore_axis_name="s")


_K2 = _EPAD // (_CH * _NSUB)
_EP2 = _EPAD // _NSUB
_CS = 64
_KG = _EP2 // _CS


def _sc_gather(hrp, hcp, ridx_g, cidx_g):
    @functools.partial(
        pl.kernel,
        out_type=(
            jax.ShapeDtypeStruct((_EPAD, _NH), f32),
            jax.ShapeDtypeStruct((_EPAD, _NH), f32),
        ),
        mesh=_sc_mesh(),
        scratch_types=[
            pltpu.VMEM((_KG // 2, _CS), i32),
            pltpu.VMEM((_CS, _NH), f32),
            pltpu.VMEM((_CS, _NH), f32),
            pltpu.VMEM_SHARED((_N, _NH), f32),
            pltpu.SemaphoreType.DMA,
            pltpu.SemaphoreType.DMA,
            pltpu.SemaphoreType.DMA,
            pltpu.SemaphoreType.DMA,
        ],
    )
    def k(hrp_h, hcp_h, ridx_h, cidx_h, hr_o, hc_o,
          idx_v, hb0, hb1, spm_tab, sg0, sg1, sw0, sw1):
        c = lax.axis_index("c")
        s = lax.axis_index("s")

        @pl.when(jnp.logical_and(c == 0, s == 0))
        def _():
            pltpu.sync_copy(hrp_h, spm_tab)

        @pl.when(jnp.logical_and(c == 1, s == 0))
        def _():
            pltpu.sync_copy(hcp_h, spm_tab)

        plsc.subcore_barrier()
        nhalf = _KG // 2
        npair = nhalf // 2

        def emit(idx_src, out_h):
            for half in range(2):
                pltpu.sync_copy(idx_src.at[s, half], idx_v)
                base = s * _EP2 + half * nhalf * _CS
                pltpu.async_copy(spm_tab.at[idx_v.at[0]], hb0, sg0)

                def pair(jj, carry):
                    j0 = jj * 2
                    off0 = base + j0 * _CS
                    off1 = off0 + _CS
                    pltpu.make_async_copy(
                        spm_tab.at[idx_v.at[j0]], hb0, sg0).wait()

                    @pl.when(jj > 0)
                    def _():
                        pltpu.make_async_copy(
                            hb1, out_h.at[pl.ds(base, _CS)], sw1).wait()

                    pltpu.async_copy(spm_tab.at[idx_v.at[j0 + 1]], hb1, sg1)
                    pltpu.async_copy(hb0, out_h.at[pl.ds(off0, _CS)], sw0)
                    pltpu.make_async_copy(
                        spm_tab.at[idx_v.at[j0 + 1]], hb1, sg1).wait()
                    pltpu.make_async_copy(
                        hb0, out_h.at[pl.ds(base, _CS)], sw0).wait()

                    @pl.when(jj < npair - 1)
                    def _():
                        pltpu.async_copy(spm_tab.at[idx_v.at[j0 + 2]], hb0, sg0)

                    pltpu.async_copy(hb1, out_h.at[pl.ds(off1, _CS)], sw1)
                    return carry

                lax.fori_loop(0, npair, pair, 0)
                pltpu.make_async_copy(
                    hb1, out_h.at[pl.ds(base, _CS)], sw1).wait()

        @pl.when(c == 0)
        def _():
            emit(ridx_h, hr_o)

        @pl.when(c == 1)
        def _():
            emit(cidx_h, hc_o)

    return k(hrp, hcp, ridx_g, cidx_g)


def _sc_xgather(xflat, ridx16, cidx16):
    @functools.partial(
        pl.kernel,
        out_type=(
            jax.ShapeDtypeStruct((_EPAD * 4,), f32),
            jax.ShapeDtypeStruct((_EPAD * 4,), f32),
        ),
        mesh=_sc_mesh(),
        scratch_types=[
            pltpu.VMEM((_K2, _CH), i32),
            pltpu.VMEM((_N * 4,), f32),
            pltpu.VMEM((_CH * 8,), f32),
            pltpu.VMEM((_CH * 8,), f32),
            pltpu.SemaphoreType.DMA,
            pltpu.SemaphoreType.DMA,
        ],
        compiler_params=pltpu.CompilerParams(needs_layout_passes=False),
    )
    def k(x_h, ridx_h, cidx_h, xr_o, xc_o, idx_v, xtab, xb0, xb1, sx0, sx1):
        c = lax.axis_index("c")
        s = lax.axis_index("s")

        @pl.when(c == 0)
        def _():
            pltpu.sync_copy(ridx_h.at[s], idx_v)

        @pl.when(c == 1)
        def _():
            pltpu.sync_copy(cidx_h.at[s], idx_v)

        pltpu.sync_copy(x_h, xtab)
        base = s * _EP2
        lanes = lax.iota(i32, 16)
        cx = 2 * _CH
        nck = _EP2 // cx

        def xgather(j, xb):
            for half in range(2):
                row = idx_v.at[2 * j + half]
                for g in range(_CH // 16):
                    iv = row[pl.ds(g * 16, 16)] * 4
                    pos = (lanes + half * _CH + g * 16) * 4
                    for kk in range(4):
                        plsc.store_scatter(xb, [pos + kk],
                                           plsc.load_gather(xtab, [iv + kk]))

        def emit(out_x):
            def pair(jj, carry):
                j0 = jj * 2
                for j, xb, sx in ((j0, xb0, sx0), (j0 + 1, xb1, sx1)):
                    @pl.when(jj > 0)
                    def _():
                        pltpu.make_async_copy(
                            xb, out_x.at[pl.ds(base * 4, cx * 4)], sx).wait()

                    xgather(j, xb)
                    pltpu.async_copy(
                        xb, out_x.at[pl.ds((base + j * cx) * 4, cx * 4)], sx)
                return carry

            lax.fori_loop(0, nck // 2, pair, 0)
            pltpu.make_async_copy(
                xb0, out_x.at[pl.ds(base * 4, cx * 4)], sx0).wait()
            xgather(nck - 1, xb0)
            pltpu.async_copy(
                xb0, out_x.at[pl.ds((base + (nck - 1) * cx) * 4, cx * 4)], sx0)
            for xb, sx in ((xb0, sx0), (xb1, sx1)):
                pltpu.make_async_copy(
                    xb, out_x.at[pl.ds(base * 4, cx * 4)], sx).wait()

        @pl.when(c == 0)
        def _():
            emit(xr_o)

        @pl.when(c == 1)
        def _():
            emit(xc_o)

    return k(xflat, ridx16, cidx16)


def _scatter_loop(src_h, base, nchunks, sidx_v, acc, mb0, mb1, sl0, sl1):
    pltpu.async_copy(src_h.at[pl.ds(base, _CS)], mb0, sl0)
    npair = nchunks // 2

    def pair(jj, carry):
        j0 = jj * 2
        off0 = base + j0 * _CS
        pltpu.make_async_copy(src_h.at[pl.ds(off0, _CS)], mb0, sl0).wait()
        pltpu.async_copy(src_h.at[pl.ds(off0 + _CS, _CS)], mb1, sl1)
        pltpu.sync_copy(mb0, acc.at[sidx_v.at[j0]], add=True)
        pltpu.make_async_copy(src_h.at[pl.ds(off0, _CS)], mb1, sl1).wait()

        @pl.when(jj < npair - 1)
        def _():
            pltpu.async_copy(src_h.at[pl.ds(off0 + 2 * _CS, _CS)], mb0, sl0)

        pltpu.sync_copy(mb1, acc.at[sidx_v.at[j0 + 1]], add=True)
        return carry

    lax.fori_loop(0, npair, pair, 0)


def _sc_scatter_mt(m, trans_pad, sidx16, z128):
    @functools.partial(
        pl.kernel,
        out_type=(
            jax.ShapeDtypeStruct((_NACC, _NH), f32),
            jax.ShapeDtypeStruct((_NACC, _NH), f32),
        ),
        mesh=_sc_mesh(),
        scratch_types=[
            pltpu.VMEM((_KG // 2, _CS), i32),
            pltpu.VMEM((_CS, _NH), f32),
            pltpu.VMEM((_CS, _NH), f32),
            pltpu.VMEM_SHARED((_NACC, _NH), f32),
            pltpu.SemaphoreType.DMA,
            pltpu.SemaphoreType.DMA,
        ],
    )
    def k(m_h, t_h, sidx_h, z_h, p_o, q_o, sidx_v, mb0, mb1, acc, sl0, sl1):
        c = lax.axis_index("c")
        s = lax.axis_index("s")

        @pl.when(s == 0)
        def _():
            pltpu.sync_copy(z_h, acc)

        plsc.subcore_barrier()

        def halves(src_h):
            for half in range(2):
                pltpu.sync_copy(sidx_h.at[s, half], sidx_v)
                base = s * _EP2 + half * (_KG // 2) * _CS
                _scatter_loop(src_h, base, _KG // 2, sidx_v, acc,
                              mb0, mb1, sl0, sl1)

        @pl.when(c == 0)
        def _():
            halves(m_h)

        @pl.when(c == 1)
        def _():
            halves(t_h)

        plsc.subcore_barrier()

        seg = _NACC // _NSUB
        start = s * seg

        @pl.when(c == 0)
        def _():
            pltpu.sync_copy(acc.at[pl.ds(start, seg)], p_o.at[pl.ds(start, seg)])

        @pl.when(c == 1)
        def _():
            pltpu.sync_copy(acc.at[pl.ds(start, seg)], q_o.at[pl.ds(start, seg)])

    return k(m, trans_pad, sidx16, z128)


def _sc_scatter_m(m, sidx, z128):
    @functools.partial(
        pl.kernel,
        out_type=(
            jax.ShapeDtypeStruct((_NACC, _NH), f32),
            jax.ShapeDtypeStruct((_NACC, _NH), f32),
        ),
        mesh=_sc_mesh(),
        scratch_types=[
            pltpu.VMEM((_EP // _CS, _CS), i32),
            pltpu.VMEM((_CS, _NH), f32),
            pltpu.VMEM((_CS, _NH), f32),
            pltpu.VMEM_SHARED((_NACC, _NH), f32),
            pltpu.SemaphoreType.DMA,
            pltpu.SemaphoreType.DMA,
        ],
    )
    def k(m_h, sidx_h, z_h, p0_o, p1_o, sidx_v, mb0, mb1, acc, sl0, sl1):
        c = lax.axis_index("c")
        s = lax.axis_index("s")
        wid = s * _NC + c

        @pl.when(s == 0)
        def _():
            pltpu.sync_copy(z_h, acc)

        pltpu.sync_copy(sidx_h.at[wid], sidx_v)
        plsc.subcore_barrier()
        base = wid * _EP
        _scatter_loop(m_h, base, _EP // _CS, sidx_v, acc, mb0, mb1, sl0, sl1)
        plsc.subcore_barrier()

        seg = _NACC // _NSUB
        start = s * seg

        @pl.when(c == 0)
        def _():
            pltpu.sync_copy(acc.at[pl.ds(start, seg)], p0_o.at[pl.ds(start, seg)])

        @pl.when(c == 1)
        def _():
            pltpu.sync_copy(acc.at[pl.ds(start, seg)], p1_o.at[pl.ds(start, seg)])

    return k(m, sidx, z128)


def _psi(t):
    return jnp.sign(t) * jnp.log(jnp.abs(t) + 1.0)


def _dot3(a, w):
    return jnp.dot(a, w, preferred_element_type=f32)


def _edge_a(hr_g, hc_g, xr, xc, ea, w4):
    def body(hr, hc, xr_r, xc_r, ea_r, w4_r, m1_o, st_o):
        b = pl.program_id(0)
        xrb = xr_r[...]
        xcb = xc_r[...]
        xd = xrb - xcb
        normsq = xd[:, 0:1] ** 2 - (xd[:, 1:2] ** 2 + xd[:, 2:3] ** 2 + xd[:, 3:4] ** 2)
        dotsq = xrb[:, 0:1] * xcb[:, 0:1] - (
            xrb[:, 1:2] * xcb[:, 1:2] + xrb[:, 2:3] * xcb[:, 2:3] + xrb[:, 3:4] * xcb[:, 3:4])
        norms = _psi(normsq)
        dots = _psi(dotsq)
        eab = ea_r[...]
        w = w4_r[...]
        eterm = (eab[:, 0:1] * w[0:1, :] + eab[:, 1:2] * w[1:2, :]
                 + norms * w[2:3, :] + dots * w[3:4, :])
        m1 = hr[...] + hc[...] + eterm
        m1_o[...] = m1
        valid = (lax.broadcasted_iota(i32, (_BE, 1), 0) + b * _BE) < _E
        m1m = jnp.where(valid, m1, 0.0)
        ssum = jnp.sum(m1m, axis=0, keepdims=True)
        ssq = jnp.sum(m1m * m1m, axis=0, keepdims=True)
        upd = jnp.concatenate([ssum, ssq, jnp.zeros((6, _NH), f32)], axis=0)

        @pl.when(b == 0)
        def _():
            st_o[...] = jnp.zeros((8, _NH), f32)

        st_o[...] += upd

    return pl.pallas_call(
        body,
        grid=(_GE,),
        in_specs=[
            pl.BlockSpec((_BE, _NH), lambda b: (b, 0)),
            pl.BlockSpec((_BE, _NH), lambda b: (b, 0)),
            pl.BlockSpec((_BE, 4), lambda b: (b, 0)),
            pl.BlockSpec((_BE, 4), lambda b: (b, 0)),
            pl.BlockSpec((_BE, 2), lambda b: (b, 0)),
            pl.BlockSpec((4, _NH), lambda b: (0, 0)),
        ],
        out_specs=[
            pl.BlockSpec((_BE, _NH), lambda b: (b, 0)),
            pl.BlockSpec((8, _NH), lambda b: (0, 0)),
        ],
        out_shape=[
            jax.ShapeDtypeStruct((_EPAD, _NH), f32),
            jax.ShapeDtypeStruct((8, _NH), f32),
        ],
    )(hr_g, hc_g, xr, xc, ea, w4)


def _edge_b(m1, stats, gamma, beta, We2, be2, Wm, bm, xr, xc, Wx1, bx1, Wx2, last):
    def body(*refs):
        if last:
            (m1_r, st_r, g_r, b_r, We2_r, be2_r, Wm_r, bm_r, m_o) = refs
        else:
            (m1_r, st_r, g_r, b_r, We2_r, be2_r, Wm_r, bm_r,
             xr_r, xc_r, Wx1_r, bx1_r, Wx2_r, m_o, tr_o) = refs
        inv = 1.0 / float(_E)
        ssum = st_r[0:1, :]
        ssq = st_r[1:2, :]
        mu = ssum * inv
        var = ssq * inv - mu * mu
        scale = g_r[...] * lax.rsqrt(var + 1e-5)
        shift = b_r[...] - mu * scale
        e1 = jnp.maximum(m1_r[...] * scale + shift, 0.0)
        m2 = jnp.maximum(_dot3(e1, We2_r[...]) + be2_r[...], 0.0)
        wgt = jax.nn.sigmoid(
            jnp.dot(m2, Wm_r[...], preferred_element_type=f32) + bm_r[...])
        mv = m2 * wgt
        m_o[...] = mv
        if not last:
            t1 = jnp.maximum(_dot3(mv, Wx1_r[...]) + bx1_r[...], 0.0)
            t = jnp.dot(t1, Wx2_r[...], preferred_element_type=f32)
            tr = (xr_r[...] - xc_r[...]) * t
            tr_o[...] = jnp.concatenate(
                [tr, jnp.zeros((_BE, _NH - 4), f32)], axis=1)

    in_specs = [
        pl.BlockSpec((_BE, _NH), lambda b: (b, 0)),
        pl.BlockSpec((8, _NH), lambda b: (0, 0)),
        pl.BlockSpec((1, _NH), lambda b: (0, 0)),
        pl.BlockSpec((1, _NH), lambda b: (0, 0)),
        pl.BlockSpec((_NH, _NH), lambda b: (0, 0)),
        pl.BlockSpec((1, _NH), lambda b: (0, 0)),
        pl.BlockSpec((_NH, 1), lambda b: (0, 0)),
        pl.BlockSpec((1, 1), lambda b: (0, 0)),
    ]
    out_specs = [pl.BlockSpec((_BE, _NH), lambda b: (b, 0))]
    out_shape = [jax.ShapeDtypeStruct((_EPAD, _NH), f32)]
    args = [m1, stats, gamma, beta, We2, be2, Wm, bm]
    if not last:
        in_specs += [
            pl.BlockSpec((_BE, 4), lambda b: (b, 0)),
            pl.BlockSpec((_BE, 4), lambda b: (b, 0)),
            pl.BlockSpec((_NH, _NH), lambda b: (0, 0)),
            pl.BlockSpec((1, _NH), lambda b: (0, 0)),
            pl.BlockSpec((_NH, 1), lambda b: (0, 0)),
        ]
        out_specs += [pl.BlockSpec((_BE, _NH), lambda b: (b, 0))]
        out_shape += [jax.ShapeDtypeStruct((_EPAD, _NH), f32)]
        args += [xr, xc, Wx1, bx1, Wx2]
    res = pl.pallas_call(
        body, grid=(_GE,), in_specs=in_specs, out_specs=out_specs,
        out_shape=out_shape)(*args)
    return res[0] if last else (res[0], res[1])


def _embed(scalars, Wemb, bemb, We1r0, We1c0):
    def body(s_r, We_r, be_r, Wr_r, Wc_r, h_o, hr_o, hc_o):
        h = jnp.dot(s_r[...], We_r[...], preferred_element_type=f32) + be_r[...]
        h_o[...] = h
        hr_o[...] = _dot3(h, Wr_r[...])
        hc_o[...] = _dot3(h, Wc_r[...])

    return pl.pallas_call(
        body,
        grid=(_GN,),
        in_specs=[
            pl.BlockSpec((_BN, 16), lambda b: (b, 0)),
            pl.BlockSpec((16, _NH), lambda b: (0, 0)),
            pl.BlockSpec((1, _NH), lambda b: (0, 0)),
            pl.BlockSpec((_NH, _NH), lambda b: (0, 0)),
            pl.BlockSpec((_NH, _NH), lambda b: (0, 0)),
        ],
        out_specs=[
            pl.BlockSpec((_BN, _NH), lambda b: (b, 0)),
            pl.BlockSpec((_BN, _NH), lambda b: (b, 0)),
            pl.BlockSpec((_BN, _NH), lambda b: (b, 0)),
        ],
        out_shape=[
            jax.ShapeDtypeStruct((_N, _NH), f32),
            jax.ShapeDtypeStruct((_N, _NH), f32),
            jax.ShapeDtypeStruct((_N, _NH), f32),
        ],
    )(scalars, Wemb, bemb, We1r0, We1c0)


def _node1(h, parts, Wh1a, Wh1b, bh1):
    np_ = len(parts)

    def body(*refs):
        h_r = refs[0]
        p_rs = refs[1:1 + np_]
        Wa_r, Wb_r, b_r, z_o, st_o = refs[1 + np_:]
        b = pl.program_id(0)
        hagg = p_rs[0][...]
        for pr in p_rs[1:]:
            hagg = hagg + pr[...]
        z = _dot3(h_r[...], Wa_r[...]) + _dot3(hagg, Wb_r[...]) + b_r[...]
        z_o[...] = z
        ssum = jnp.sum(z, axis=0, keepdims=True)
        ssq = jnp.sum(z * z, axis=0, keepdims=True)
        upd = jnp.concatenate([ssum, ssq, jnp.zeros((6, _NH), f32)], axis=0)

        @pl.when(b == 0)
        def _():
            st_o[...] = jnp.zeros((8, _NH), f32)

        st_o[...] += upd

    return pl.pallas_call(
        body,
        grid=(_GN,),
        in_specs=(
            [pl.BlockSpec((_BN, _NH), lambda b: (b, 0))] * (1 + np_)
            + [
                pl.BlockSpec((_NH, _NH), lambda b: (0, 0)),
                pl.BlockSpec((_NH, _NH), lambda b: (0, 0)),
                pl.BlockSpec((1, _NH), lambda b: (0, 0)),
            ]
        ),
        out_specs=[
            pl.BlockSpec((_BN, _NH), lambda b: (b, 0)),
            pl.BlockSpec((8, _NH), lambda b: (0, 0)),
        ],
        out_shape=[
            jax.ShapeDtypeStruct((_N, _NH), f32),
            jax.ShapeDtypeStruct((8, _NH), f32),
        ],
    )(h, *parts, Wh1a, Wh1b, bh1)


def _node2(z, stats, gh, bh, Wh2, bh2, h, x, xq, We1r_n, We1c_n, last):
    def body(*refs):
        if last:
            (z_r, st_r, g_r, b_r, W2_r, b2_r, h_r, h_o) = refs
        else:
            (z_r, st_r, g_r, b_r, W2_r, b2_r, h_r, x_r, xq_r,
             Wr_r, Wc_r, h_o, hr_o, hc_o, x_o) = refs
        inv = 1.0 / float(_N)
        mu = st_r[0:1, :] * inv
        var = st_r[1:2, :] * inv - mu * mu
        scale = g_r[...] * lax.rsqrt(var + 1e-5)
        shift = b_r[...] - mu * scale
        hh = jnp.maximum(z_r[...] * scale + shift, 0.0)
        hn = h_r[...] + _dot3(hh, W2_r[...]) + b2_r[...]
        h_o[...] = hn
        if not last:
            hr_o[...] = _dot3(hn, Wr_r[...])
            hc_o[...] = _dot3(hn, Wc_r[...])
            x_o[...] = x_r[...] + xq_r[...]

    in_specs = [
        pl.BlockSpec((_BN, _NH), lambda b: (b, 0)),
        pl.BlockSpec((8, _NH), lambda b: (0, 0)),
        pl.BlockSpec((1, _NH), lambda b: (0, 0)),
        pl.BlockSpec((1, _NH), lambda b: (0, 0)),
        pl.BlockSpec((_NH, _NH), lambda b: (0, 0)),
        pl.BlockSpec((1, _NH), lambda b: (0, 0)),
        pl.BlockSpec((_BN, _NH), lambda b: (b, 0)),
    ]
    out_specs = [pl.BlockSpec((_BN, _NH), lambda b: (b, 0))]
    out_shape = [jax.ShapeDtypeStruct((_N, _NH), f32)]
    args = [z, stats, gh, bh, Wh2, bh2, h]
    if not last:
        in_specs += [
            pl.BlockSpec((_BN, 4), lambda b: (b, 0)),
            pl.BlockSpec((_BN, 4), lambda b: (b, 0)),
            pl.BlockSpec((_NH, _NH), lambda b: (0, 0)),
            pl.BlockSpec((_NH, _NH), lambda b: (0, 0)),
        ]
        out_specs += [
            pl.BlockSpec((_BN, _NH), lambda b: (b, 0)),
            pl.BlockSpec((_BN, _NH), lambda b: (b, 0)),
            pl.BlockSpec((_BN, 4), lambda b: (b, 0)),
        ]
        out_shape += [
            jax.ShapeDtypeStruct((_N, _NH), f32),
            jax.ShapeDtypeStruct((_N, _NH), f32),
            jax.ShapeDtypeStruct((_N, 4), f32),
        ]
        args += [x, xq, We1r_n, We1c_n]
    res = pl.pallas_call(
        body, grid=(_GN,), in_specs=in_specs, out_specs=out_specs,
        out_shape=out_shape)(*args)
    return res[0] if last else (res[0], res[1], res[2], res[3])


def _head(x, h, wxo, who, bo):
    def body(x_r, h_r, wx_r, wh_r, b_r, o_o):
        o = (jnp.dot(x_r[...], wx_r[...], preferred_element_type=f32)
             + jnp.dot(h_r[...], wh_r[...], preferred_element_type=f32) + b_r[...])
        o_o[...] = jax.nn.sigmoid(o)

    return pl.pallas_call(
        body,
        grid=(_GN,),
        in_specs=[
            pl.BlockSpec((_BN, 4), lambda b: (b, 0)),
            pl.BlockSpec((_BN, _NH), lambda b: (b, 0)),
            pl.BlockSpec((4, 1), lambda b: (0, 0)),
            pl.BlockSpec((_NH, 1), lambda b: (0, 0)),
            pl.BlockSpec((1, 1), lambda b: (0, 0)),
        ],
        out_specs=pl.BlockSpec((_BN, 1), lambda b: (b, 0)),
        out_shape=jax.ShapeDtypeStruct((_N, 1), f32),
    )(x, h, wxo, who, bo)


def _r2(v):
    return v.reshape(1, -1)


def kernel(scalars, x, edge_index, edge_attr, params):
    row = edge_index[0].astype(i32)
    col = edge_index[1].astype(i32)
    padz = jnp.zeros((_EPAD - _E,), i32)
    rpad = jnp.concatenate([row, padz])
    cpad = jnp.concatenate([col, padz])
    ridx16 = rpad.reshape(_NSUB, _K2, _CH)
    cidx16 = cpad.reshape(_NSUB, _K2, _CH)
    ridx_g = rpad.reshape(_NSUB, 2, _KG // 2, _CS)
    cidx_g = cpad.reshape(_NSUB, 2, _KG // 2, _CS)
    spad = jnp.concatenate([row, jnp.full((_EPAD - _E,), _N, i32)])
    sidx = spad.reshape(_W, _EP // _CS, _CS)
    sidx16 = spad.reshape(_NSUB, 2, _KG // 2, _CS)
    ea_pad = jnp.concatenate(
        [edge_attr, jnp.zeros((_EPAD - _E, 2), f32)], axis=0)
    z128 = jnp.zeros((_NACC, _NH), f32)

    We1 = params['We1']
    Wh1 = params['Wh1']

    h, hrp, hcp = _embed(scalars, params['Wemb'], _r2(params['bemb']),
                         We1[0, :_NH], We1[0, _NH:2 * _NH])

    for i in range(_L):
        last = i == _L - 1
        hr_g, hc_g = _sc_gather(hrp, hcp, ridx_g, cidx_g)
        xrf, xcf = _sc_xgather(x.reshape(-1), ridx16, cidx16)
        xr = xrf.reshape(_EPAD, 4)
        xc = xcf.reshape(_EPAD, 4)
        m1, st_e = _edge_a(hr_g, hc_g, xr, xc, ea_pad, We1[i, 2 * _NH:])
        if last:
            m = _edge_b(m1, st_e, _r2(params['bn_e_g'][i]), _r2(params['bn_e_b'][i]),
                        params['We2'][i], _r2(params['be2'][i]),
                        params['Wm'][i], _r2(params['bm'][i]),
                        None, None, None, None, None, True)
            p0, p1 = _sc_scatter_m(m, sidx, z128)
            parts = [p0[:_N], p1[:_N]]
        else:
            m, trans = _edge_b(m1, st_e, _r2(params['bn_e_g'][i]), _r2(params['bn_e_b'][i]),
                               params['We2'][i], _r2(params['be2'][i]),
                               params['Wm'][i], _r2(params['bm'][i]),
                               xr, xc, params['Wx1'][i], _r2(params['bx1'][i]),
                               params['Wx2'][i], False)
            p, q = _sc_scatter_mt(m, trans, sidx16, z128)
            parts = [p[:_N]]
        z, st_n = _node1(h, parts, Wh1[i, :_NH], Wh1[i, _NH:],
                         _r2(params['bh1'][i]))
        if last:
            h = _node2(z, st_n, _r2(params['bn_h_g'][i]), _r2(params['bn_h_b'][i]),
                       params['Wh2'][i], _r2(params['bh2'][i]), h,
                       None, None, None, None, True)
        else:
            h, hrp, hcp, x = _node2(
                z, st_n, _r2(params['bn_h_g'][i]), _r2(params['bn_h_b'][i]),
                params['Wh2'][i], _r2(params['bh2'][i]), h,
                x, q[:_N, 0:4], We1[i + 1, :_NH], We1[i + 1, _NH:2 * _NH],
                False)

    return _head(x, h, params['Wout'][:4], params['Wout'][4:],
                 _r2(params['bout']))
